# baseline (speedup 1.0000x reference)
import functools

import jax
import jax.numpy as jnp
from jax import lax
from jax.experimental import pallas as pl
from jax.experimental.pallas import tpu as pltpu
from jax.experimental.pallas import tpu_sc as plsc

NUM_CLASSES = 100000
D = 64
B = 16384
ALPHA = 0.5

NC = 2
NS = 16
L = 16

CH = 12576
PASSES = 4
ACC_ROWS = CH + 8
BT = B // NS
LB = B // (NC * NS)
CSL = CH // NS
NCH = BT // 128

_f32 = jnp.float32
_i32 = jnp.int32


def _sc_center_loss(features, labels, centers):
    mesh = plsc.VectorSubcoreMesh(core_axis_name="c", subcore_axis_name="s")

    @functools.partial(
        pl.kernel,
        out_type=(
            jax.ShapeDtypeStruct((NUM_CLASSES, D), _f32),
            jax.ShapeDtypeStruct((NC * 8, L), _f32),
        ),
        mesh=mesh,
        compiler_params=pltpu.CompilerParams(use_tc_tiling_on_sc=False),
        scratch_types=[
            pltpu.VMEM_SHARED((ACC_ROWS, D), _f32),
            pltpu.VMEM_SHARED((ACC_ROWS, L), _f32),
            pltpu.VMEM_SHARED((NS, L), _f32),
            pltpu.VMEM((256, D), _f32),
            pltpu.VMEM((128, D), _f32),
            pltpu.VMEM((256, D), _f32),
            pltpu.VMEM((128, L), _f32),
            pltpu.VMEM((128, L), _f32),
            pltpu.VMEM((BT,), _i32),
            pltpu.VMEM((NCH, 128), _i32),
            pltpu.VMEM((NCH, 128), _i32),
            pltpu.VMEM((NCH, 128), _i32),
            pltpu.VMEM((128,), _i32),
            pltpu.VMEM((L,), _f32),
            pltpu.VMEM((NS, L), _f32),
        ] + [pltpu.SemaphoreType.DMA] * 11,
    )
    def body(feat_hbm, lab_hbm, cent_hbm, out_hbm, loss_hbm,
             sums_sh, cnts_sh, loss_sh,
             fbuf, sbuf, cbuf, cntb, ones,
             lbuf, iadd, idxg, idxl, lidx, accv, lall,
             sem_f0, sem_f1, sem_g0, sem_g1, sem_sc0, sem_sc1,
             sem_z, sem_cr0, sem_cr1, sem_cw0, sem_cw1):
        c = lax.axis_index("c")
        s = lax.axis_index("s")
        zero16 = jnp.zeros((L,), _f32)
        one16 = jnp.ones((L,), _f32)
        zero16i = jnp.zeros((L,), _i32)
        sem_f = [sem_f0, sem_f1]
        sem_g = [sem_g0, sem_g1]
        sem_sc = [sem_sc0, sem_sc1]
        sem_cr = [sem_cr0, sem_cr1]
        sem_cw = [sem_cw0, sem_cw1]

        def _init(i, _):
            ones[i, pl.ds(0, L)] = one16
            for g in range(4):
                cbuf[i, pl.ds(g * L, L)] = zero16
            cntb[i, pl.ds(0, L)] = zero16
            return 0

        lax.fori_loop(0, 128, _init, 0)

        cz = cbuf.at[pl.ds(0, 128)]

        def _zero_list():
            lst = []
            for t in range(6):
                zr = (s + t * NS) * 128
                lst.append((cz, sums_sh.at[pl.ds(zr, 128)]))
                lst.append((cntb, cnts_sh.at[pl.ds(zr, 128)]))
            return lst

        def _zero_extra_issue(fn):
            @pl.when(s < 2)
            def _():
                zr = (96 + s) * 128
                fn(cz, sums_sh.at[pl.ds(zr, 128)])
                fn(cntb, cnts_sh.at[pl.ds(zr, 128)])
            @pl.when(s == 2)
            def _():
                fn(cbuf.at[pl.ds(0, 32)], sums_sh.at[pl.ds(12544, 32)])
                fn(cntb.at[pl.ds(0, 32)], cnts_sh.at[pl.ds(12544, 32)])

        def _zero_issue():
            for src, dst in _zero_list():
                pltpu.async_copy(src, dst, sem_z)
            _zero_extra_issue(lambda a, b: pltpu.async_copy(a, b, sem_z))

        def _zero_drain():
            for src, dst in _zero_list():
                pltpu.make_async_copy(src, dst, sem_z).wait()
            _zero_extra_issue(
                lambda a, b: pltpu.make_async_copy(a, b, sem_z).wait())

        _zero_issue()

        w = s * NC + c
        lb0 = w * LB
        pltpu.sync_copy(lab_hbm.at[pl.ds(lb0, LB)], lbuf.at[pl.ds(0, LB)])

        def _lft_issue(j):
            pltpu.async_copy(feat_hbm.at[pl.ds(lb0 + j * 128, 128)],
                             fbuf.at[pl.ds((j & 1) * 128, 128)],
                             sem_f[j & 1])

        def _lft_wait(j):
            pltpu.make_async_copy(feat_hbm.at[pl.ds(lb0 + j * 128, 128)],
                                  fbuf.at[pl.ds((j & 1) * 128, 128)],
                                  sem_f[j & 1]).wait()

        _lft_issue(0)
        acc = zero16
        for j in range(LB // 128):
            if j + 1 < LB // 128:
                _lft_issue(j + 1)
            for t in range(8):
                lidx[pl.ds(t * L, L)] = lbuf[pl.ds(j * 128 + t * L, L)]
            pltpu.sync_copy(cent_hbm.at[lidx], sbuf)
            _lft_wait(j)
            par = j & 1

            def _lacc(r, a):
                for g in range(4):
                    dv = (fbuf[par * 128 + r, pl.ds(g * L, L)]
                          - sbuf[r, pl.ds(g * L, L)])
                    a = a + dv * dv
                return a

            acc = lax.fori_loop(0, 128, _lacc, acc)
        accv[pl.ds(0, L)] = acc
        pltpu.sync_copy(accv, loss_sh.at[s])

        pltpu.sync_copy(lab_hbm.at[pl.ds(s * BT, BT)], lbuf)
        _zero_drain()
        plsc.subcore_barrier()

        @pl.when(s == 0)
        def _():
            pltpu.sync_copy(loss_sh, lall)
            red = zero16
            for t in range(NS):
                red = red + lall[t, pl.ds(0, L)]
            accv[pl.ds(0, L)] = red
            pltpu.sync_copy(accv, loss_hbm.at[c * 8])

        bb = s * BT
        for p in range(PASSES):
            lo = (2 * p + c) * CH
            lo_v = jnp.full((L,), lo, _i32)
            hi_v = lo_v + CH
            ch_v = jnp.full((L,), CH, _i32)

            if p > 0:
                def _zinit(i, _):
                    for g in range(4):
                        cbuf[i, pl.ds(g * L, L)] = zero16
                    cntb[i, pl.ds(0, L)] = zero16
                    return 0

                lax.fori_loop(0, 128, _zinit, 0)
                _zero_issue()
                _zero_drain()
                plsc.subcore_barrier()

            for k in range(BT // L):
                lv = lbuf[pl.ds(k * L, L)]
                inr = jnp.logical_and(lv >= lo_v, lv < hi_v)
                row = k // 8
                off = (k % 8) * L
                iadd[row, pl.ds(off, L)] = jnp.where(inr, lv - lo_v, ch_v)
                idxg[row, pl.ds(off, L)] = jnp.where(inr, lv, lo_v)
                idxl[row, pl.ds(off, L)] = jnp.where(inr, lv - lo_v, zero16i)

            cbase = jnp.minimum(lo + s * CSL, NUM_CLASSES - CSL)
            CCH = [(i * 64, 64) for i in range(12)] + [(768, 18)]
            SUBMAP = [[0, 1], [2, 3], [4, 5], [6, 7], [8, 9], [10, 11],
                      [12], []]

            def _cp_rd(u):
                off, n = CCH[u]
                return (cent_hbm.at[pl.ds(cbase + off, n)],
                        sbuf.at[pl.ds((u & 1) * 64, n)], sem_cr[u & 1])

            def _cp_wr(u):
                off, n = CCH[u]
                return (sbuf.at[pl.ds((u & 1) * 64, n)],
                        out_hbm.at[pl.ds(cbase + off, n)], sem_cw[u & 1])

            def _cp_sub(u):
                if u >= 1:
                    pltpu.make_async_copy(*_cp_wr(u - 1)).wait()
                pltpu.make_async_copy(*_cp_rd(u)).wait()
                pltpu.async_copy(*_cp_wr(u))
                if u + 1 < len(CCH):
                    pltpu.async_copy(*_cp_rd(u + 1))

            def _ft_issue(h):
                pltpu.async_copy(feat_hbm.at[pl.ds(bb + h * 128, 128)],
                                 fbuf.at[pl.ds((h & 1) * 128, 128)],
                                 sem_f[h & 1])

            def _ft_wait(h):
                pltpu.make_async_copy(feat_hbm.at[pl.ds(bb + h * 128, 128)],
                                      fbuf.at[pl.ds((h & 1) * 128, 128)],
                                      sem_f[h & 1]).wait()

            pltpu.async_copy(*_cp_rd(0))
            _ft_issue(0)
            for h in range(NCH):
                for u in SUBMAP[h]:
                    _cp_sub(u)
                if h + 1 < NCH:
                    _ft_issue(h + 1)
                _ft_wait(h)
                pltpu.sync_copy(fbuf.at[pl.ds((h & 1) * 128, 128)],
                                sums_sh.at[iadd.at[h]], add=True)
                pltpu.sync_copy(ones, cnts_sh.at[iadd.at[h]], add=True)
            pltpu.make_async_copy(*_cp_wr(len(CCH) - 1)).wait()
            plsc.subcore_barrier()

            def _cg_issue(j):
                pltpu.async_copy(cent_hbm.at[idxg.at[j]],
                                 cbuf.at[pl.ds((j & 1) * 128, 128)],
                                 sem_g[j & 1])

            def _cg_wait(j):
                pltpu.make_async_copy(cent_hbm.at[idxg.at[j]],
                                      cbuf.at[pl.ds((j & 1) * 128, 128)],
                                      sem_g[j & 1]).wait()

            def _sc_issue(j):
                pltpu.async_copy(cbuf.at[pl.ds((j & 1) * 128, 128)],
                                 out_hbm.at[idxg.at[j]], sem_sc[j & 1])

            def _sc_wait(j):
                pltpu.make_async_copy(cbuf.at[pl.ds((j & 1) * 128, 128)],
                                      out_hbm.at[idxg.at[j]],
                                      sem_sc[j & 1]).wait()

            _cg_issue(0)
            for j in range(NCH):
                if j + 1 < NCH:
                    if j >= 1:
                        _sc_wait(j - 1)
                    _cg_issue(j + 1)
                pltpu.sync_copy(sums_sh.at[idxl.at[j]], sbuf)
                pltpu.sync_copy(cnts_sh.at[idxl.at[j]], cntb)
                _cg_wait(j)
                par = j & 1

                def _upd(r, _):
                    cnt = cntb[r, pl.ds(0, L)]
                    pred = cnt > 0.0
                    rv = ALPHA / jnp.maximum(cnt, 1.0)
                    for g in range(4):
                        sv = sbuf[r, pl.ds(g * L, L)]
                        cv = cbuf[par * 128 + r, pl.ds(g * L, L)]
                        cbuf[par * 128 + r, pl.ds(g * L, L)] = jnp.where(
                            pred, (1.0 - ALPHA) * cv + sv * rv, cv)
                    return 0

                lax.fori_loop(0, 128, _upd, 0)
                _sc_issue(j)
            _sc_wait(NCH - 2)
            _sc_wait(NCH - 1)
            plsc.subcore_barrier()

    return body(features, labels, centers)


def kernel(features, labels, centers):
    out, loss_part = _sc_center_loss(features, labels, centers)
    loss = jnp.sum(loss_part[0] + loss_part[8]) / jnp.float32(B * D)
    return loss, out


# speedup vs baseline: 5.2001x; 5.2001x over previous
import functools

import jax
import jax.numpy as jnp
from jax import lax
from jax.experimental import pallas as pl
from jax.experimental.pallas import tpu as pltpu
from jax.experimental.pallas import tpu_sc as plsc

NUM_CLASSES = 100000
D = 64
B = 16384
ALPHA = 0.5

NC = 2
NS = 16
L = 16

CH = 16768
PASSES = 3
ACC_ROWS = CH + 8
BT = B // NS
LB = B // (NC * NS)
CSL = CH // NS

_f32 = jnp.float32
_i32 = jnp.int32


def _sc_center_loss(features, labels, centers):
    mesh = plsc.VectorSubcoreMesh(core_axis_name="c", subcore_axis_name="s")

    @functools.partial(
        pl.kernel,
        out_type=(
            jax.ShapeDtypeStruct((NUM_CLASSES, D), _f32),
            jax.ShapeDtypeStruct((NC * 8, L), _f32),
        ),
        mesh=mesh,
        compiler_params=pltpu.CompilerParams(use_tc_tiling_on_sc=False),
        scratch_types=[
            pltpu.VMEM_SHARED((ACC_ROWS, D), _f32),
            pltpu.VMEM_SHARED((ACC_ROWS, L), _f32),
            pltpu.VMEM_SHARED((NS, L), _f32),
            pltpu.VMEM((128, D), _f32),
            pltpu.VMEM((128, D), _f32),
            pltpu.VMEM((128, D), _f32),
            pltpu.VMEM((128, L), _f32),
            pltpu.VMEM((128, L), _f32),
            pltpu.VMEM((BT,), _i32),
            pltpu.VMEM((8, 128), _i32),
            pltpu.VMEM((128,), _i32),
            pltpu.VMEM((L,), _f32),
            pltpu.VMEM((NS, L), _f32),
        ],
    )
    def body(feat_hbm, lab_hbm, cent_hbm, out_hbm, loss_hbm,
             sums_sh, cnts_sh, loss_sh,
             fbuf, sbuf, cbuf, cntb, ones,
             lbuf, iadd, idx128, accv, lall):
        c = lax.axis_index("c")
        s = lax.axis_index("s")
        zero16 = jnp.zeros((L,), _f32)
        one16 = jnp.ones((L,), _f32)

        def _init(i, _):
            ones[i, pl.ds(0, L)] = one16
            return 0

        lax.fori_loop(0, 128, _init, 0)

        w = s * NC + c
        lb0 = w * LB
        pltpu.sync_copy(lab_hbm.at[pl.ds(lb0, LB)], lbuf.at[pl.ds(0, LB)])
        acc = zero16
        for j in range(LB // 128):
            pltpu.sync_copy(feat_hbm.at[pl.ds(lb0 + j * 128, 128)], fbuf)
            for t in range(8):
                idx128[pl.ds(t * L, L)] = lbuf[pl.ds(j * 128 + t * L, L)]
            pltpu.sync_copy(cent_hbm.at[idx128], sbuf)

            def _lacc(r, a):
                for g in range(4):
                    dv = (fbuf[r, pl.ds(g * L, L)]
                          - sbuf[r, pl.ds(g * L, L)])
                    a = a + dv * dv
                return a

            acc = lax.fori_loop(0, 128, _lacc, acc)
        accv[pl.ds(0, L)] = acc
        pltpu.sync_copy(accv, loss_sh.at[s])
        plsc.subcore_barrier()

        @pl.when(s == 0)
        def _():
            pltpu.sync_copy(loss_sh, lall)
            red = zero16
            for t in range(NS):
                red = red + lall[t, pl.ds(0, L)]
            accv[pl.ds(0, L)] = red
            pltpu.sync_copy(accv, loss_hbm.at[c * 8])

        for p in range(PASSES):
            lo = (2 * p + c) * CH
            lo_v = jnp.full((L,), lo, _i32)
            hi_v = lo_v + CH
            ch_v = jnp.full((L,), CH, _i32)

            def _zinit(i, _):
                for g in range(4):
                    cbuf[i, pl.ds(g * L, L)] = zero16
                cntb[i, pl.ds(0, L)] = zero16
                return 0

            lax.fori_loop(0, 128, _zinit, 0)

            def _zero_chunk(zrow):
                pltpu.sync_copy(cbuf, sums_sh.at[pl.ds(zrow, 128)])
                pltpu.sync_copy(cntb, cnts_sh.at[pl.ds(zrow, 128)])

            for t in range(8):
                _zero_chunk((s + t * NS) * 128)
            @pl.when(s < 3)
            def _():
                _zero_chunk((s + 8 * NS) * 128)
            plsc.subcore_barrier()

            bb = s * BT
            pltpu.sync_copy(lab_hbm.at[pl.ds(bb, BT)], lbuf)
            for k in range(BT // L):
                lv = lbuf[pl.ds(k * L, L)]
                inr = jnp.logical_and(lv >= lo_v, lv < hi_v)
                iadd[k // 8, pl.ds((k % 8) * L, L)] = jnp.where(
                    inr, lv - lo_v, ch_v)

            for h in range(8):
                pltpu.sync_copy(feat_hbm.at[pl.ds(bb + h * 128, 128)], fbuf)
                pltpu.sync_copy(fbuf, sums_sh.at[iadd.at[h]], add=True)
                pltpu.sync_copy(ones, cnts_sh.at[iadd.at[h]], add=True)
            plsc.subcore_barrier()

            cbase = jnp.minimum(lo + s * CSL, NUM_CLASSES - CSL)
            sbase = cbase - lo
            for u in range(9):
                off = u * 128
                n = 128 if u < 8 else CSL - 1024
                pltpu.sync_copy(cent_hbm.at[pl.ds(cbase + off, n)],
                                cbuf.at[pl.ds(0, n)])
                pltpu.sync_copy(sums_sh.at[pl.ds(sbase + off, n)],
                                sbuf.at[pl.ds(0, n)])
                pltpu.sync_copy(cnts_sh.at[pl.ds(sbase + off, n)],
                                cntb.at[pl.ds(0, n)])

                def _upd(r, _):
                    cnt = cntb[r, pl.ds(0, L)]
                    pred = cnt > 0.0
                    rv = ALPHA / jnp.maximum(cnt, 1.0)
                    for g in range(4):
                        sv = sbuf[r, pl.ds(g * L, L)]
                        cv = cbuf[r, pl.ds(g * L, L)]
                        cbuf[r, pl.ds(g * L, L)] = jnp.where(
                            pred, (1.0 - ALPHA) * cv + sv * rv, cv)
                    return 0

                lax.fori_loop(0, n, _upd, 0)
                pltpu.sync_copy(cbuf.at[pl.ds(0, n)],
                                out_hbm.at[pl.ds(cbase + off, n)])
            plsc.subcore_barrier()

    return body(features, labels, centers)


def kernel(features, labels, centers):
    out, loss_part = _sc_center_loss(features, labels, centers)
    loss = jnp.sum(loss_part[0] + loss_part[8]) / jnp.float32(B * D)
    return loss, out


# speedup vs baseline: 5.5460x; 1.0665x over previous
import functools

import jax
import jax.numpy as jnp
from jax import lax
from jax.experimental import pallas as pl
from jax.experimental.pallas import tpu as pltpu
from jax.experimental.pallas import tpu_sc as plsc

NUM_CLASSES = 100000
D = 64
B = 16384
ALPHA = 0.5

NC = 2
NS = 16
L = 16

CH = 16768
PASSES = 3
ACC_ROWS = CH + 8
BT = B // NS
LB = B // (NC * NS)
CSL = CH // NS

_f32 = jnp.float32
_i32 = jnp.int32


def _sc_center_loss(features, labels, centers):
    mesh = plsc.VectorSubcoreMesh(core_axis_name="c", subcore_axis_name="s")

    @functools.partial(
        pl.kernel,
        out_type=(
            jax.ShapeDtypeStruct((NUM_CLASSES, D), _f32),
            jax.ShapeDtypeStruct((NC * 8, L), _f32),
        ),
        mesh=mesh,
        compiler_params=pltpu.CompilerParams(use_tc_tiling_on_sc=False),
        scratch_types=[
            pltpu.VMEM_SHARED((ACC_ROWS, D), _f32),
            pltpu.VMEM_SHARED((ACC_ROWS, L), _f32),
            pltpu.VMEM_SHARED((NS, L), _f32),
            pltpu.VMEM((128, D), _f32),
            pltpu.VMEM((128, D), _f32),
            pltpu.VMEM((256, D), _f32),
            pltpu.VMEM((128, L), _f32),
            pltpu.VMEM((128, L), _f32),
            pltpu.VMEM((BT,), _i32),
            pltpu.VMEM((8, 128), _i32),
            pltpu.VMEM((128,), _i32),
            pltpu.VMEM((L,), _f32),
        ] + [pltpu.SemaphoreType.DMA] * 6,
    )
    def body(feat_hbm, lab_hbm, cent_hbm, out_hbm, loss_hbm,
             sums_sh, cnts_sh, loss_sh,
             fbuf, sbuf, cbuf, cntb, ones,
             lbuf, iadd, idx128, accv,
             sem_f0, sem_f1, sem_cr0, sem_cr1, sem_cw0, sem_cw1):
        c = lax.axis_index("c")
        s = lax.axis_index("s")
        zero16 = jnp.zeros((L,), _f32)
        one16 = jnp.ones((L,), _f32)
        sem_f = [sem_f0, sem_f1]
        sem_cr = [sem_cr0, sem_cr1]
        sem_cw = [sem_cw0, sem_cw1]

        def _init(i, _):
            ones[i, pl.ds(0, L)] = one16
            return 0

        lax.fori_loop(0, 128, _init, 0)

        w = s * NC + c
        lb0 = w * LB
        pltpu.sync_copy(lab_hbm.at[pl.ds(lb0, LB)], lbuf.at[pl.ds(0, LB)])
        acc = zero16
        for j in range(LB // 128):
            pltpu.sync_copy(feat_hbm.at[pl.ds(lb0 + j * 128, 128)],
                            fbuf.at[pl.ds(0, 128)])
            for t in range(8):
                idx128[pl.ds(t * L, L)] = lbuf[pl.ds(j * 128 + t * L, L)]
            pltpu.sync_copy(cent_hbm.at[idx128], sbuf)

            def _lacc(r, a):
                for g in range(4):
                    dv = (fbuf[r, pl.ds(g * L, L)]
                          - sbuf[r, pl.ds(g * L, L)])
                    a = a + dv * dv
                return a

            acc = lax.fori_loop(0, 128, _lacc, acc)
        accv[pl.ds(0, L)] = acc
        pltpu.sync_copy(accv, loss_sh.at[s])
        plsc.subcore_barrier()

        @pl.when(s == 0)
        def _():
            pltpu.sync_copy(loss_sh, cntb.at[pl.ds(0, NS)])
            red = zero16
            for t in range(NS):
                red = red + cntb[t, pl.ds(0, L)]
            accv[pl.ds(0, L)] = red
            pltpu.sync_copy(accv, loss_hbm.at[c * 8])

        bb = s * BT
        pltpu.sync_copy(lab_hbm.at[pl.ds(bb, BT)], lbuf)

        for p in range(PASSES):
            lo = (2 * p + c) * CH
            lo_v = jnp.full((L,), lo, _i32)
            hi_v = lo_v + CH
            ch_v = jnp.full((L,), CH, _i32)

            def _zinit(i, _):
                for g in range(4):
                    cbuf[i, pl.ds(g * L, L)] = zero16
                cntb[i, pl.ds(0, L)] = zero16
                return 0

            lax.fori_loop(0, 128, _zinit, 0)

            def _zero_chunk(zrow):
                pltpu.sync_copy(cbuf.at[pl.ds(0, 128)],
                                sums_sh.at[pl.ds(zrow, 128)])
                pltpu.sync_copy(cntb, cnts_sh.at[pl.ds(zrow, 128)])

            for t in range(8):
                _zero_chunk((s + t * NS) * 128)
            @pl.when(s < 3)
            def _():
                _zero_chunk((s + 8 * NS) * 128)
            plsc.subcore_barrier()

            for k in range(BT // L):
                lv = lbuf[pl.ds(k * L, L)]
                inr = jnp.logical_and(lv >= lo_v, lv < hi_v)
                iadd[k // 8, pl.ds((k % 8) * L, L)] = jnp.where(
                    inr, lv - lo_v, ch_v)

            def _ft_issue(h):
                pltpu.async_copy(feat_hbm.at[pl.ds(bb + h * 128, 128)],
                                 cbuf.at[pl.ds((h & 1) * 128, 128)],
                                 sem_f[h & 1])

            def _ft_wait(h):
                pltpu.make_async_copy(feat_hbm.at[pl.ds(bb + h * 128, 128)],
                                      cbuf.at[pl.ds((h & 1) * 128, 128)],
                                      sem_f[h & 1]).wait()

            _ft_issue(0)
            for h in range(8):
                if h + 1 < 8:
                    _ft_issue(h + 1)
                _ft_wait(h)
                pltpu.sync_copy(cbuf.at[pl.ds((h & 1) * 128, 128)],
                                sums_sh.at[iadd.at[h]], add=True)
                pltpu.sync_copy(ones, cnts_sh.at[iadd.at[h]], add=True)
            plsc.subcore_barrier()

            cbase = jnp.minimum(lo + s * CSL, NUM_CLASSES - CSL)
            sbase = cbase - lo
            NU = 9
            ULEN = [128] * 8 + [CSL - 1024]

            def _cr(u):
                return (cent_hbm.at[pl.ds(cbase + u * 128, ULEN[u])],
                        cbuf.at[pl.ds((u & 1) * 128, ULEN[u])],
                        sem_cr[u & 1])

            def _cw(u):
                return (cbuf.at[pl.ds((u & 1) * 128, ULEN[u])],
                        out_hbm.at[pl.ds(cbase + u * 128, ULEN[u])],
                        sem_cw[u & 1])

            pltpu.async_copy(*_cr(0))
            for u in range(NU):
                if u + 1 < NU:
                    if u >= 1:
                        pltpu.make_async_copy(*_cw(u - 1)).wait()
                    pltpu.async_copy(*_cr(u + 1))
                n = ULEN[u]
                pltpu.sync_copy(sums_sh.at[pl.ds(sbase + u * 128, n)],
                                sbuf.at[pl.ds(0, n)])
                pltpu.sync_copy(cnts_sh.at[pl.ds(sbase + u * 128, n)],
                                cntb.at[pl.ds(0, n)])
                pltpu.make_async_copy(*_cr(u)).wait()
                par = u & 1

                def _upd(r, _):
                    cnt = cntb[r, pl.ds(0, L)]
                    pred = cnt > 0.0
                    rv = ALPHA / jnp.maximum(cnt, 1.0)
                    for g in range(4):
                        sv = sbuf[r, pl.ds(g * L, L)]
                        cv = cbuf[par * 128 + r, pl.ds(g * L, L)]
                        cbuf[par * 128 + r, pl.ds(g * L, L)] = jnp.where(
                            pred, (1.0 - ALPHA) * cv + sv * rv, cv)
                    return 0

                lax.fori_loop(0, n, _upd, 0)
                pltpu.async_copy(*_cw(u))
            pltpu.make_async_copy(*_cw(NU - 2)).wait()
            pltpu.make_async_copy(*_cw(NU - 1)).wait()
            plsc.subcore_barrier()

    return body(features, labels, centers)


def kernel(features, labels, centers):
    out, loss_part = _sc_center_loss(features, labels, centers)
    loss = jnp.sum(loss_part[0] + loss_part[8]) / jnp.float32(B * D)
    return loss, out
